# baseline (speedup 1.0000x reference)
import functools

import jax
import jax.numpy as jnp
from jax import lax
from jax.experimental import pallas as pl
from jax.experimental.pallas import tpu as pltpu, tpu_sc as plsc

_N = 66048
_E = 1056768
_F = 64
_CH = 32
_HALF = 16
_NPG = 258
_HID = 128

_NC = 2
_NS = 16

_SB = 128
_KB = 32
_CHUNK = _SB * _KB
_EROWS = _E // _SB
_NCHUNK = _EROWS // _KB
_TROWS = _N // _SB


def _tc1_body(x_ref, wc0_ref, wc1_ref, xc0_ref, yt_ref):
    xv = x_ref[...]
    xc0_ref[...] = jnp.dot(xv, wc0_ref[...], preferred_element_type=jnp.float32)
    yt_ref[...] = lax.dot_general(
        wc1_ref[...], xv,
        dimension_numbers=(((0,), (1,)), ((), ())),
        preferred_element_type=jnp.float32).astype(jnp.bfloat16)


_TC1_ROWS = _N // 4

_tc1 = pl.pallas_call(
    _tc1_body,
    grid=(4,),
    in_specs=[
        pl.BlockSpec((_TC1_ROWS, _F), lambda i: (i, 0)),
        pl.BlockSpec((_F, _CH), lambda i: (0, 0)),
        pl.BlockSpec((_F, _CH), lambda i: (0, 0)),
    ],
    out_specs=[
        pl.BlockSpec((_TC1_ROWS, _CH), lambda i: (i, 0)),
        pl.BlockSpec((_CH, _TC1_ROWS), lambda i: (0, i)),
    ],
    out_shape=[
        jax.ShapeDtypeStruct((_N, _CH), jnp.float32),
        jax.ShapeDtypeStruct((_CH, _N), jnp.bfloat16),
    ],
)


_ZB = 8256
_NPAIR = _EROWS // (2 * _KB)


@functools.partial(
    pl.kernel,
    out_type=jax.ShapeDtypeStruct((_CH * _N,), jnp.float32),
    mesh=plsc.VectorSubcoreMesh(core_axis_name="c", subcore_axis_name="s"),
    compiler_params=pltpu.CompilerParams(needs_layout_passes=False),
    scratch_types=[
        pltpu.VMEM((_N // 2,), jnp.int32),
        pltpu.VMEM((_N,), jnp.float32),
        pltpu.VMEM((_CHUNK,), jnp.int32),
        pltpu.VMEM((_CHUNK,), jnp.int32),
        pltpu.VMEM((_CHUNK,), jnp.int32),
        pltpu.VMEM((_CHUNK,), jnp.int32),
        pltpu.VMEM((_CHUNK,), jnp.float32),
        pltpu.VMEM((_CHUNK,), jnp.float32),
        pltpu.SemaphoreType.DMA,
        pltpu.SemaphoreType.DMA,
    ],
)
def _sc_segsum(yt1, src2, dst2, w2, out,
               tq, acc, srcbA, srcbB, dstbA, dstbB, wbA, wbB, lsemA, lsemB):
    c = lax.axis_index("c")
    s = lax.axis_index("s")
    q = c * _NS + s

    pltpu.sync_copy(yt1.at[pl.ds(q * (_N // 2), _N // 2)], tq)

    def _zero_body(k, carry):
        acc[pl.ds(k * 16, 16)] = jnp.zeros((16,), jnp.float32)
        return carry

    lax.fori_loop(0, _N // 16, _zero_body, 0)

    def _fire_loads(row0, srcb, dstb, wb, lsem):
        e0 = row0 * _SB
        pltpu.async_copy(src2.at[pl.ds(e0, _CHUNK)], srcb, lsem)
        pltpu.async_copy(dst2.at[pl.ds(e0, _CHUNK)], dstb, lsem)
        pltpu.async_copy(w2.at[pl.ds(e0, _CHUNK)], wb, lsem)

    def _wait_loads(srcb, dstb, wb, lsem):
        pltpu.make_async_copy(src2.at[pl.ds(0, _CHUNK)], srcb, lsem).wait()
        pltpu.make_async_copy(dst2.at[pl.ds(0, _CHUNK)], dstb, lsem).wait()
        pltpu.make_async_copy(w2.at[pl.ds(0, _CHUNK)], wb, lsem).wait()

    _fire_loads(0, srcbA, dstbA, wbA, lsemA)
    _fire_loads(_KB, srcbB, dstbB, wbB, lsemB)

    def _do_chunk(next_row, srcb, dstb, wb, lsem):
        _wait_loads(srcb, dstb, wb, lsem)

        @plsc.parallel_loop(0, _CHUNK // 16, unroll=16)
        def _grp_body(k):
            srcv = srcb[pl.ds(k * 16, 16)]
            wv = wb[pl.ds(k * 16, 16)]
            dv = dstb[pl.ds(k * 16, 16)]
            word = plsc.load_gather(tq, [srcv >> 1])
            sh = (srcv & 1) << 4
            vals = plsc.bitcast((word >> sh) << 16, jnp.float32)
            plsc.addupdate_scatter(acc, [dv], vals * wv)

        @pl.when(next_row < _EROWS)
        def _():
            _fire_loads(next_row, srcb, dstb, wb, lsem)

    def _pair_body(i, carry):
        row0 = 2 * i * _KB
        _do_chunk(row0 + 2 * _KB, srcbA, dstbA, wbA, lsemA)
        _do_chunk(row0 + 3 * _KB, srcbB, dstbB, wbB, lsemB)
        return carry

    lax.fori_loop(0, _NPAIR, _pair_body, 0)

    pltpu.sync_copy(acc, out.at[pl.ds(q * _N, _N)])


def _tc2a_body(xc0_ref, t1_ref, bc_ref, h_ref):
    sv = xc0_ref[...] + t1_ref[...] + bc_ref[...]
    h_ref[...] = jnp.where(sv > 0, sv, jnp.exp(sv) - 1.0)


_tc2a = pl.pallas_call(
    _tc2a_body,
    grid=(4,),
    in_specs=[
        pl.BlockSpec((_TC1_ROWS, _CH), lambda i: (i, 0)),
        pl.BlockSpec((_TC1_ROWS, _CH), lambda i: (i, 0)),
        pl.BlockSpec((1, _CH), lambda i: (0, 0)),
    ],
    out_specs=pl.BlockSpec((_TC1_ROWS, _CH), lambda i: (i, 0)),
    out_shape=jax.ShapeDtypeStruct((_N, _CH), jnp.float32),
)


def _mlp_body(h_ref, w1_ref, b1_ref, w2_ref, b2_ref, w3_ref, b3_ref,
              w4t_ref, b4_ref, o_ref):
    a = jnp.dot(h_ref[...], w1_ref[...], preferred_element_type=jnp.float32)
    a = jnp.maximum(a + b1_ref[...], 0.0)
    a = jnp.dot(a, w2_ref[...], preferred_element_type=jnp.float32)
    a = jnp.maximum(a + b2_ref[...], 0.0)
    a = jnp.dot(a, w3_ref[...], preferred_element_type=jnp.float32)
    a = jnp.maximum(a + b3_ref[...], 0.0)
    z = jnp.sum(a * w4t_ref[...], axis=1, keepdims=True) + b4_ref[...]
    o_ref[...] = jax.nn.sigmoid(z)


_NG = _N // _NPG

_mlp = pl.pallas_call(
    _mlp_body,
    out_shape=jax.ShapeDtypeStruct((_NG, 1), jnp.float32),
)


def kernel(x, edge_index, edge_weight, i, Wc0, Wc1, bc,
           W1, b1, W2, b2, W3, b3, W4, b4):
    del i
    src2 = edge_index[0]
    dst2 = edge_index[1]
    w2 = edge_weight

    xc0, yt = _tc1(x, Wc0, Wc1)

    yt_i32 = lax.bitcast_convert_type(
        yt.reshape(_CH, _N // 2, 2), jnp.int32)
    t1_flat = _sc_segsum(yt_i32.reshape(_CH * (_N // 2)), src2, dst2, w2)
    parts = t1_flat.reshape(2, _CH // 2, _N)
    t1 = (parts[0] + parts[1]).T

    h = _tc2a(xc0, t1, bc.reshape(1, _CH))
    h = h.reshape(_NG, _NPG * _CH)

    out = _mlp(h, W1, b1.reshape(1, _HID),
               W2, b2.reshape(1, _HID // 2),
               W3, b3.reshape(1, _HID // 4),
               W4.reshape(1, _HID // 4), b4.reshape(1, 1))
    return out


# speedup vs baseline: 1.0254x; 1.0254x over previous
import functools

import jax
import jax.numpy as jnp
from jax import lax
from jax.experimental import pallas as pl
from jax.experimental.pallas import tpu as pltpu, tpu_sc as plsc

_N = 66048
_E = 1056768
_F = 64
_CH = 32
_HALF = 16
_NPG = 258
_HID = 128

_NC = 2
_NS = 16

_SB = 128
_KB = 32
_CHUNK = _SB * _KB
_EROWS = _E // _SB
_NCHUNK = _EROWS // _KB
_TROWS = _N // _SB


def _tc1_body(x_ref, wc0_ref, wc1_ref, xc0_ref, yt_ref):
    xv = x_ref[...]
    xc0_ref[...] = jnp.dot(xv, wc0_ref[...], preferred_element_type=jnp.float32)
    yt_ref[...] = lax.dot_general(
        wc1_ref[...], xv,
        dimension_numbers=(((0,), (1,)), ((), ())),
        preferred_element_type=jnp.float32).astype(jnp.bfloat16)


_TC1_ROWS = _N // 4

_tc1 = pl.pallas_call(
    _tc1_body,
    grid=(4,),
    in_specs=[
        pl.BlockSpec((_TC1_ROWS, _F), lambda i: (i, 0)),
        pl.BlockSpec((_F, _CH), lambda i: (0, 0)),
        pl.BlockSpec((_F, _CH), lambda i: (0, 0)),
    ],
    out_specs=[
        pl.BlockSpec((_TC1_ROWS, _CH), lambda i: (i, 0)),
        pl.BlockSpec((_CH, _TC1_ROWS), lambda i: (0, i)),
    ],
    out_shape=[
        jax.ShapeDtypeStruct((_N, _CH), jnp.float32),
        jax.ShapeDtypeStruct((_CH, _N), jnp.bfloat16),
    ],
)


_ZB = 8256
_NPAIR = _EROWS // (2 * _KB)


@functools.partial(
    pl.kernel,
    out_type=jax.ShapeDtypeStruct((_CH * _N,), jnp.float32),
    mesh=plsc.VectorSubcoreMesh(core_axis_name="c", subcore_axis_name="s"),
    compiler_params=pltpu.CompilerParams(needs_layout_passes=False),
    scratch_types=[
        pltpu.VMEM((_N // 2,), jnp.int32),
        pltpu.VMEM((_N,), jnp.float32),
        pltpu.VMEM((_CHUNK,), jnp.int32),
        pltpu.VMEM((_CHUNK,), jnp.int32),
        pltpu.VMEM((_CHUNK,), jnp.int32),
        pltpu.VMEM((_CHUNK,), jnp.int32),
        pltpu.VMEM((_CHUNK,), jnp.float32),
        pltpu.VMEM((_CHUNK,), jnp.float32),
        pltpu.SemaphoreType.DMA,
        pltpu.SemaphoreType.DMA,
    ],
)
def _sc_segsum(yt1, src2, dst2, w2, out,
               tq, acc, srcbA, srcbB, dstbA, dstbB, wbA, wbB, lsemA, lsemB):
    c = lax.axis_index("c")
    s = lax.axis_index("s")
    q = c * _NS + s

    pltpu.sync_copy(yt1.at[pl.ds(q * (_N // 2), _N // 2)], tq)

    def _zero_body(k, carry):
        acc[pl.ds(k * 16, 16)] = jnp.zeros((16,), jnp.float32)
        return carry

    lax.fori_loop(0, _N // 16, _zero_body, 0)

    def _fire_loads(row0, srcb, dstb, wb, lsem):
        e0 = row0 * _SB
        pltpu.async_copy(src2.at[pl.ds(e0, _CHUNK)], srcb, lsem)
        pltpu.async_copy(dst2.at[pl.ds(e0, _CHUNK)], dstb, lsem)
        pltpu.async_copy(w2.at[pl.ds(e0, _CHUNK)], wb, lsem)

    def _wait_loads(srcb, dstb, wb, lsem):
        pltpu.make_async_copy(src2.at[pl.ds(0, _CHUNK)], srcb, lsem).wait()
        pltpu.make_async_copy(dst2.at[pl.ds(0, _CHUNK)], dstb, lsem).wait()
        pltpu.make_async_copy(w2.at[pl.ds(0, _CHUNK)], wb, lsem).wait()

    _fire_loads(0, srcbA, dstbA, wbA, lsemA)
    _fire_loads(_KB, srcbB, dstbB, wbB, lsemB)

    def _do_chunk(next_row, srcb, dstb, wb, lsem):
        _wait_loads(srcb, dstb, wb, lsem)

        @plsc.parallel_loop(0, _CHUNK // 16, unroll=16)
        def _grp_body(k):
            srcv = srcb[pl.ds(k * 16, 16)]
            wv = wb[pl.ds(k * 16, 16)]
            dv = dstb[pl.ds(k * 16, 16)]
            word = plsc.load_gather(tq, [srcv >> 1])
            sh = (srcv & 1) << 4
            vals = plsc.bitcast((word >> sh) << 16, jnp.float32)
            plsc.addupdate_scatter(acc, [dv], vals * wv)

        @pl.when(next_row < _EROWS)
        def _():
            _fire_loads(next_row, srcb, dstb, wb, lsem)

    def _pair_body(i, carry):
        row0 = 2 * i * _KB
        _do_chunk(row0 + 2 * _KB, srcbA, dstbA, wbA, lsemA)
        _do_chunk(row0 + 3 * _KB, srcbB, dstbB, wbB, lsemB)
        return carry

    lax.fori_loop(0, _NPAIR, _pair_body, 0)

    pltpu.sync_copy(acc, out.at[pl.ds(q * _N, _N)])


def _tc2a_body(xc0_ref, t1t_ref, bc_ref, h_ref):
    sv = xc0_ref[...] + t1t_ref[...].T + bc_ref[...]
    h_ref[...] = jnp.where(sv > 0, sv, jnp.exp(sv) - 1.0)


_tc2a = pl.pallas_call(
    _tc2a_body,
    grid=(4,),
    in_specs=[
        pl.BlockSpec((_TC1_ROWS, _CH), lambda i: (i, 0)),
        pl.BlockSpec((_CH, _TC1_ROWS), lambda i: (0, i)),
        pl.BlockSpec((1, _CH), lambda i: (0, 0)),
    ],
    out_specs=pl.BlockSpec((_TC1_ROWS, _CH), lambda i: (i, 0)),
    out_shape=jax.ShapeDtypeStruct((_N, _CH), jnp.float32),
)


def _mlp_body(h_ref, w1_ref, b1_ref, w2_ref, b2_ref, w3_ref, b3_ref,
              w4t_ref, b4_ref, o_ref):
    a = jnp.dot(h_ref[...], w1_ref[...], preferred_element_type=jnp.float32)
    a = jnp.maximum(a + b1_ref[...], 0.0)
    a = jnp.dot(a, w2_ref[...], preferred_element_type=jnp.float32)
    a = jnp.maximum(a + b2_ref[...], 0.0)
    a = jnp.dot(a, w3_ref[...], preferred_element_type=jnp.float32)
    a = jnp.maximum(a + b3_ref[...], 0.0)
    z = jnp.sum(a * w4t_ref[...], axis=1, keepdims=True) + b4_ref[...]
    o_ref[...] = jax.nn.sigmoid(z)


_NG = _N // _NPG

_mlp = pl.pallas_call(
    _mlp_body,
    out_shape=jax.ShapeDtypeStruct((_NG, 1), jnp.float32),
)


def kernel(x, edge_index, edge_weight, i, Wc0, Wc1, bc,
           W1, b1, W2, b2, W3, b3, W4, b4):
    del i
    src2 = edge_index[0]
    dst2 = edge_index[1]
    w2 = edge_weight

    xc0, yt = _tc1(x, Wc0, Wc1)

    yt_i32 = lax.bitcast_convert_type(
        yt.reshape(_CH, _N // 2, 2), jnp.int32)
    t1_flat = _sc_segsum(yt_i32.reshape(_CH * (_N // 2)), src2, dst2, w2)
    parts = t1_flat.reshape(2, _CH // 2, _N)
    t1 = (parts[0] + parts[1]).T

    h = _tc2a(xc0, t1, bc.reshape(1, _CH))
    h = h.reshape(_NG, _NPG * _CH)

    out = _mlp(h, W1, b1.reshape(1, _HID),
               W2, b2.reshape(1, _HID // 2),
               W3, b3.reshape(1, _HID // 4),
               W4.reshape(1, _HID // 4), b4.reshape(1, 1))
    return out
